# baseline (speedup 1.0000x reference)
import functools

import jax
import jax.numpy as jnp
from jax import lax
from jax.experimental import pallas as pl
from jax.experimental.pallas import tpu as pltpu
from jax.experimental.pallas import tpu_sc as plsc

_BLOCK = 512

_NC = 2
_NS = 16
_NW = _NC * _NS
_GCHUNK = 128


def _argmin_kernel(x_ref, cb_ref, x2_ref, c2_ref, idx_ref, cbm2_ref):
    i = pl.program_id(0)

    @pl.when(i == 0)
    def _():
        cbm2_ref[...] = cb_ref[...] * -2.0

    x = x_ref[...]
    x2 = x2_ref[...]
    xcm2 = jax.lax.dot_general(
        x, cbm2_ref[...], (((1,), (1,)), ((), ())),
        preferred_element_type=jnp.float32)
    dist = jnp.sqrt(jnp.maximum(x2 + c2_ref[...] + xcm2, 0.0))
    k = dist.shape[1]
    m = jnp.min(dist, axis=-1, keepdims=True)
    iota = jax.lax.broadcasted_iota(jnp.int32, dist.shape, 1)
    idx = jnp.min(jnp.where(dist == m, iota, k), axis=-1)
    idx_ref[...] = idx.astype(jnp.int32).reshape(1, 1, idx.shape[0])


def _tc_argmin(x, codebook):
    n, d = x.shape
    k = codebook.shape[0]
    grid = n // _BLOCK
    idx3 = pl.pallas_call(
        _argmin_kernel,
        grid=(grid,),
        in_specs=[
            pl.BlockSpec((_BLOCK, d), lambda i: (i, 0)),
            pl.BlockSpec((k, d), lambda i: (0, 0)),
            pl.BlockSpec((_BLOCK, 1), lambda i: (i, 0)),
            pl.BlockSpec((1, k), lambda i: (0, 0)),
        ],
        out_specs=pl.BlockSpec((1, 1, _BLOCK), lambda i: (i, 0, 0)),
        out_shape=jax.ShapeDtypeStruct((grid, 1, _BLOCK), jnp.int32),
        scratch_shapes=[
            pltpu.VMEM((k, d), jnp.float32),
        ],
    )(x, codebook,
      jnp.sum(x * x, axis=-1, keepdims=True),
      jnp.sum(codebook * codebook, axis=-1)[None, :])
    return idx3.reshape(n)


def _sc_gather(table, idx2d, n, d):
    rows_per_w = n // _NW
    ichunks = rows_per_w // _GCHUNK
    mesh = plsc.VectorSubcoreMesh(
        core_axis_name="c", subcore_axis_name="s",
        num_cores=_NC, num_subcores=_NS)

    @functools.partial(
        pl.kernel, mesh=mesh,
        out_type=jax.ShapeDtypeStruct((n, d), jnp.float32),
        compiler_params=pltpu.CompilerParams(use_tc_tiling_on_sc=False),
        scratch_types=[
            pltpu.VMEM((ichunks, _GCHUNK), jnp.int32),
            pltpu.VMEM((rows_per_w, d), jnp.float32),
            pltpu.SemaphoreType.DMA,
        ],
    )
    def gk(table_hbm, idx_hbm, out_hbm, idx_v, rows_v, sem):
        wid = lax.axis_index("s") * _NC + lax.axis_index("c")
        pltpu.sync_copy(idx_hbm.at[pl.ds(wid * ichunks, ichunks)], idx_v)
        copies = []
        for j in range(ichunks):
            copies.append(pltpu.async_copy(
                table_hbm.at[idx_v.at[j]],
                rows_v.at[pl.ds(j * _GCHUNK, _GCHUNK)],
                sem))
        for c in copies:
            c.wait()
        pltpu.sync_copy(rows_v, out_hbm.at[pl.ds(wid * rows_per_w,
                                                 rows_per_w)])

    return gk(table, idx2d)


def kernel(x, codebook):
    n, d = x.shape
    idx = _tc_argmin(x, codebook)
    q = _sc_gather(codebook, idx.reshape(n // _GCHUNK, _GCHUNK), n, d)
    return q, idx
